# baseline (speedup 1.0000x reference)
import functools

import jax
import jax.numpy as jnp
from jax import lax
from jax.experimental import pallas as pl
from jax.experimental.pallas import tpu as pltpu
from jax.experimental.pallas import tpu_sc as plsc

_NUM_WORKERS = 32
_CHUNK = 65536


def _sc_copy_body(x_hbm, o_hbm, buf, in_sems, out_sems):
    n_total = x_hbm.shape[0]
    per_worker = n_total // _NUM_WORKERS
    n_chunks = per_worker // _CHUNK
    wid = lax.axis_index("s") * 2 + lax.axis_index("c")
    base = wid * per_worker

    def in_copy(i, b):
        return pltpu.make_async_copy(
            x_hbm.at[pl.ds(base + i * _CHUNK, _CHUNK)], buf.at[b], in_sems.at[b]
        )

    def out_copy(i, b):
        return pltpu.make_async_copy(
            buf.at[b], o_hbm.at[pl.ds(base + i * _CHUNK, _CHUNK)], out_sems.at[b]
        )

    for i in range(n_chunks):
        c_in = in_copy(i, 0)
        c_in.start()
        c_in.wait()
        c_out = out_copy(i, 0)
        c_out.start()
        c_out.wait()


def kernel(x):
    B, C, N = x.shape
    n_total = B * C * N
    x_flat = x.reshape(n_total)
    mesh = plsc.VectorSubcoreMesh(core_axis_name="c", subcore_axis_name="s")
    out = pl.kernel(
        _sc_copy_body,
        out_type=jax.ShapeDtypeStruct((n_total,), x.dtype),
        mesh=mesh,
        scratch_types=[
            pltpu.VMEM((1, _CHUNK), jnp.float32),
            pltpu.SemaphoreType.DMA((1,)),
            pltpu.SemaphoreType.DMA((1,)),
        ],
    )(x_flat)
    return out.reshape(B, C * 4, N // 4)


# speedup vs baseline: 1.6278x; 1.6278x over previous
import jax
import jax.numpy as jnp
from jax.experimental import pallas as pl
from jax.experimental.pallas import tpu as pltpu

_SLAB = 256
_NBUF = 8


def _copy_body(x_ref, o_ref, bufs, in_sems, out_sems):
    n_slabs = x_ref.shape[0] // _SLAB

    def in_copy(i):
        b = i % _NBUF
        return pltpu.make_async_copy(
            x_ref.at[pl.ds(i * _SLAB, _SLAB)], bufs.at[b], in_sems.at[b]
        )

    def out_copy(i):
        b = i % _NBUF
        return pltpu.make_async_copy(
            bufs.at[b], o_ref.at[pl.ds(i * _SLAB, _SLAB)], out_sems.at[b]
        )

    for i in range(_NBUF):
        in_copy(i).start()
    for i in range(_NBUF):
        in_copy(i).wait()
        out_copy(i).start()
    for i in range(_NBUF, n_slabs):
        out_copy(i - _NBUF).wait()
        in_copy(i).start()
    for i in range(_NBUF, n_slabs):
        in_copy(i).wait()
        out_copy(i).start()
    for i in range(n_slabs - _NBUF, n_slabs):
        out_copy(i).wait()


def kernel(x):
    B, C, N = x.shape
    total_rows = B * C
    x2 = x.reshape(total_rows, N)
    out = pl.pallas_call(
        _copy_body,
        in_specs=[pl.BlockSpec(memory_space=pl.ANY)],
        out_specs=pl.BlockSpec(memory_space=pl.ANY),
        out_shape=jax.ShapeDtypeStruct((total_rows, N), x.dtype),
        scratch_shapes=[
            pltpu.VMEM((_NBUF, _SLAB, N), jnp.float32),
            pltpu.SemaphoreType.DMA((_NBUF,)),
            pltpu.SemaphoreType.DMA((_NBUF,)),
        ],
    )(x2)
    return out.reshape(B, C * 4, N // 4)
